# Initial kernel scaffold; baseline (speedup 1.0000x reference)
import jax
import jax.numpy as jnp
from jax.experimental import pallas as pl


def kernel(flat, cu_seqlens, W1, b1, W2, b2):
    raise NotImplementedError("write your pallas kernel here")


# speedup vs baseline: 11.3690x; 11.3690x over previous
import functools

import jax
import jax.numpy as jnp
from jax.experimental import pallas as pl
from jax.experimental.pallas import tpu as pltpu

B = 16
TOTAL = 16384
NELEM = 256
HID = 512
LAT = 128
TILE = 1024


def _fused_kernel(x_ref, w1_ref, b1_ref, w2_ref, b2_ref, starts_ref, ends_ref,
                  scale_ref, nonempty_ref, out_ref, acc_ref):
    i = pl.program_id(0)
    nsteps = pl.num_programs(0)

    x = x_ref[...]
    h = jnp.maximum(
        jnp.dot(x, w1_ref[...], preferred_element_type=jnp.float32)
        + b1_ref[...], 0.0)

    rows = i * TILE + jax.lax.broadcasted_iota(jnp.int32, (TILE, 1), 0)
    onehot = ((rows >= starts_ref[...]) & (rows < ends_ref[...])).astype(
        jnp.float32)
    part = jax.lax.dot_general(
        onehot, h, (((0,), (0,)), ((), ())),
        preferred_element_type=jnp.float32)

    @pl.when(i == 0)
    def _():
        acc_ref[...] = part

    @pl.when(i > 0)
    def _():
        acc_ref[...] += part

    @pl.when(i == nsteps - 1)
    def _():
        pooled = acc_ref[...] * scale_ref[...]
        z = (jnp.dot(pooled, w2_ref[...], preferred_element_type=jnp.float32)
             + b2_ref[...])
        out_ref[...] = z * nonempty_ref[...]


@functools.partial(jax.jit, static_argnames=())
def kernel(flat, cu_seqlens, W1, b1, W2, b2):
    cu = cu_seqlens.astype(jnp.int32)
    starts = cu[:-1].reshape(1, B)
    ends = cu[1:].reshape(1, B)
    counts = (ends - starts).astype(jnp.float32).reshape(B, 1)
    nonempty = (counts > 0).astype(jnp.float32)
    scale = nonempty / jnp.maximum(counts, 1.0)

    b1r = b1.reshape(1, HID)
    b2r = b2.reshape(1, LAT)

    nsteps = TOTAL // TILE
    return pl.pallas_call(
        _fused_kernel,
        grid=(nsteps,),
        in_specs=[
            pl.BlockSpec((TILE, NELEM), lambda i: (i, 0)),
            pl.BlockSpec((NELEM, HID), lambda i: (0, 0)),
            pl.BlockSpec((1, HID), lambda i: (0, 0)),
            pl.BlockSpec((HID, LAT), lambda i: (0, 0)),
            pl.BlockSpec((1, LAT), lambda i: (0, 0)),
            pl.BlockSpec((1, B), lambda i: (0, 0)),
            pl.BlockSpec((1, B), lambda i: (0, 0)),
            pl.BlockSpec((B, 1), lambda i: (0, 0)),
            pl.BlockSpec((B, 1), lambda i: (0, 0)),
        ],
        out_specs=pl.BlockSpec((B, LAT), lambda i: (0, 0)),
        out_shape=jax.ShapeDtypeStruct((B, LAT), jnp.float32),
        scratch_shapes=[pltpu.VMEM((B, HID), jnp.float32)],
        compiler_params=pltpu.CompilerParams(
            dimension_semantics=("arbitrary",)),
    )(flat, W1, b1r, W2, b2r, starts, ends, scale, nonempty)


# speedup vs baseline: 11.4147x; 1.0040x over previous
import functools

import jax
import jax.numpy as jnp
from jax.experimental import pallas as pl
from jax.experimental.pallas import tpu as pltpu

B = 16
TOTAL = 16384
NELEM = 256
HID = 512
LAT = 128
TILE = 1024


def _fused_kernel(x_ref, w1_ref, b1_ref, w2_ref, b2_ref, starts_ref, ends_ref,
                  scale_ref, nonempty_ref, out_ref, acc_ref):
    i = pl.program_id(0)
    nsteps = pl.num_programs(0)

    x = x_ref[...].astype(jnp.bfloat16)
    h = jnp.maximum(
        jnp.dot(x, w1_ref[...].astype(jnp.bfloat16),
                preferred_element_type=jnp.float32)
        + b1_ref[...], 0.0)

    rows = i * TILE + jax.lax.broadcasted_iota(jnp.int32, (TILE, 1), 0)
    onehot = ((rows >= starts_ref[...]) & (rows < ends_ref[...])).astype(
        jnp.bfloat16)
    part = jax.lax.dot_general(
        onehot, h.astype(jnp.bfloat16), (((0,), (0,)), ((), ())),
        preferred_element_type=jnp.float32)

    @pl.when(i == 0)
    def _():
        acc_ref[...] = part

    @pl.when(i > 0)
    def _():
        acc_ref[...] += part

    @pl.when(i == nsteps - 1)
    def _():
        pooled = acc_ref[...] * scale_ref[...]
        z = (jnp.dot(pooled, w2_ref[...], preferred_element_type=jnp.float32)
             + b2_ref[...])
        out_ref[...] = z * nonempty_ref[...]


@functools.partial(jax.jit, static_argnames=())
def kernel(flat, cu_seqlens, W1, b1, W2, b2):
    cu = cu_seqlens.astype(jnp.int32)
    starts = cu[:-1].reshape(1, B)
    ends = cu[1:].reshape(1, B)
    counts = (ends - starts).astype(jnp.float32).reshape(B, 1)
    nonempty = (counts > 0).astype(jnp.float32)
    scale = nonempty / jnp.maximum(counts, 1.0)

    b1r = b1.reshape(1, HID)
    b2r = b2.reshape(1, LAT)

    nsteps = TOTAL // TILE
    return pl.pallas_call(
        _fused_kernel,
        grid=(nsteps,),
        in_specs=[
            pl.BlockSpec((TILE, NELEM), lambda i: (i, 0)),
            pl.BlockSpec((NELEM, HID), lambda i: (0, 0)),
            pl.BlockSpec((1, HID), lambda i: (0, 0)),
            pl.BlockSpec((HID, LAT), lambda i: (0, 0)),
            pl.BlockSpec((1, LAT), lambda i: (0, 0)),
            pl.BlockSpec((1, B), lambda i: (0, 0)),
            pl.BlockSpec((1, B), lambda i: (0, 0)),
            pl.BlockSpec((B, 1), lambda i: (0, 0)),
            pl.BlockSpec((B, 1), lambda i: (0, 0)),
        ],
        out_specs=pl.BlockSpec((B, LAT), lambda i: (0, 0)),
        out_shape=jax.ShapeDtypeStruct((B, LAT), jnp.float32),
        scratch_shapes=[pltpu.VMEM((B, HID), jnp.float32)],
        compiler_params=pltpu.CompilerParams(
            dimension_semantics=("arbitrary",)),
    )(flat, W1, b1r, W2, b2r, starts, ends, scale, nonempty)


# speedup vs baseline: 14.1287x; 1.2378x over previous
import functools

import jax
import jax.numpy as jnp
from jax.experimental import pallas as pl
from jax.experimental.pallas import tpu as pltpu

B = 16
TOTAL = 16384
NELEM = 256
HID = 512
LAT = 128
TILE = 2048


def _fused_kernel(x_ref, w1_ref, b1_ref, w2_ref, b2_ref, starts_ref, ends_ref,
                  scale_ref, nonempty_ref, out_ref, acc_ref):
    i = pl.program_id(0)
    nsteps = pl.num_programs(0)

    x = x_ref[...].astype(jnp.bfloat16)
    h = jnp.maximum(
        jnp.dot(x, w1_ref[...].astype(jnp.bfloat16),
                preferred_element_type=jnp.float32)
        + b1_ref[...], 0.0)

    rows = i * TILE + jax.lax.broadcasted_iota(jnp.int32, (TILE, 1), 0)
    onehot = ((rows >= starts_ref[...]) & (rows < ends_ref[...])).astype(
        jnp.float32)
    part = jax.lax.dot_general(
        onehot, h, (((0,), (0,)), ((), ())),
        preferred_element_type=jnp.float32)

    @pl.when(i == 0)
    def _():
        acc_ref[...] = part

    @pl.when(i > 0)
    def _():
        acc_ref[...] += part

    @pl.when(i == nsteps - 1)
    def _():
        pooled = acc_ref[...] * scale_ref[...]
        z = (jnp.dot(pooled, w2_ref[...], preferred_element_type=jnp.float32)
             + b2_ref[...])
        out_ref[...] = z * nonempty_ref[...]


@functools.partial(jax.jit, static_argnames=())
def kernel(flat, cu_seqlens, W1, b1, W2, b2):
    cu = cu_seqlens.astype(jnp.int32)
    starts = cu[:-1].reshape(1, B)
    ends = cu[1:].reshape(1, B)
    counts = (ends - starts).astype(jnp.float32).reshape(B, 1)
    nonempty = (counts > 0).astype(jnp.float32)
    scale = nonempty / jnp.maximum(counts, 1.0)

    b1r = b1.reshape(1, HID)
    b2r = b2.reshape(1, LAT)

    nsteps = TOTAL // TILE
    return pl.pallas_call(
        _fused_kernel,
        grid=(nsteps,),
        in_specs=[
            pl.BlockSpec((TILE, NELEM), lambda i: (i, 0)),
            pl.BlockSpec((NELEM, HID), lambda i: (0, 0)),
            pl.BlockSpec((1, HID), lambda i: (0, 0)),
            pl.BlockSpec((HID, LAT), lambda i: (0, 0)),
            pl.BlockSpec((1, LAT), lambda i: (0, 0)),
            pl.BlockSpec((1, B), lambda i: (0, 0)),
            pl.BlockSpec((1, B), lambda i: (0, 0)),
            pl.BlockSpec((B, 1), lambda i: (0, 0)),
            pl.BlockSpec((B, 1), lambda i: (0, 0)),
        ],
        out_specs=pl.BlockSpec((B, LAT), lambda i: (0, 0)),
        out_shape=jax.ShapeDtypeStruct((B, LAT), jnp.float32),
        scratch_shapes=[pltpu.VMEM((B, HID), jnp.float32)],
        compiler_params=pltpu.CompilerParams(
            dimension_semantics=("arbitrary",)),
    )(flat, W1, b1r, W2, b2r, starts, ends, scale, nonempty)


# speedup vs baseline: 17.7265x; 1.2546x over previous
import functools

import jax
import jax.numpy as jnp
from jax.experimental import pallas as pl
from jax.experimental.pallas import tpu as pltpu

B = 16
TOTAL = 16384
NELEM = 256
HID = 512
LAT = 128
TILE = 2048


def _fused_kernel(cu_ref, x_ref, w1_ref, b1_ref, w2_ref, b2_ref, out_ref,
                  acc_ref, w1bf_ref, bounds_ref):
    i = pl.program_id(0)
    nsteps = pl.num_programs(0)

    @pl.when(i == 0)
    def _():
        w1bf_ref[...] = w1_ref[...].astype(jnp.bfloat16)
        lane = jax.lax.broadcasted_iota(jnp.int32, (1, B), 1)
        sv = jnp.zeros((1, B), jnp.int32)
        ev = jnp.zeros((1, B), jnp.int32)
        for s in range(B):
            sv = jnp.where(lane == s, cu_ref[s], sv)
            ev = jnp.where(lane == s, cu_ref[s + 1], ev)
        bounds_ref[0:1, :] = sv
        bounds_ref[1:2, :] = ev

    x = x_ref[...].astype(jnp.bfloat16)
    h = jnp.maximum(
        jnp.dot(x, w1bf_ref[...], preferred_element_type=jnp.float32)
        + b1_ref[...], 0.0)

    rows = i * TILE + jax.lax.broadcasted_iota(jnp.int32, (TILE, 1), 0)
    onehot = ((rows >= bounds_ref[0:1, :]) & (rows < bounds_ref[1:2, :])
              ).astype(jnp.float32)
    part = jax.lax.dot_general(
        onehot, h, (((0,), (0,)), ((), ())),
        preferred_element_type=jnp.float32)

    @pl.when(i == 0)
    def _():
        acc_ref[...] = part

    @pl.when(i > 0)
    def _():
        acc_ref[...] += part

    @pl.when(i == nsteps - 1)
    def _():
        sub = jax.lax.broadcasted_iota(jnp.int32, (B, 1), 0)
        cnt = jnp.zeros((B, 1), jnp.int32)
        for s in range(B):
            cnt = jnp.where(sub == s, cu_ref[s + 1] - cu_ref[s], cnt)
        cntf = cnt.astype(jnp.float32)
        nonempty = (cntf > 0).astype(jnp.float32)
        scale = nonempty / jnp.maximum(cntf, 1.0)
        pooled = acc_ref[...] * scale
        z = (jnp.dot(pooled, w2_ref[...], preferred_element_type=jnp.float32)
             + b2_ref[...])
        out_ref[...] = z * nonempty


@functools.partial(jax.jit, static_argnames=())
def kernel(flat, cu_seqlens, W1, b1, W2, b2):
    b1r = b1.reshape(1, HID)
    b2r = b2.reshape(1, LAT)

    nsteps = TOTAL // TILE
    grid_spec = pltpu.PrefetchScalarGridSpec(
        num_scalar_prefetch=1,
        grid=(nsteps,),
        in_specs=[
            pl.BlockSpec((TILE, NELEM), lambda i, cu: (i, 0)),
            pl.BlockSpec((NELEM, HID), lambda i, cu: (0, 0)),
            pl.BlockSpec((1, HID), lambda i, cu: (0, 0)),
            pl.BlockSpec((HID, LAT), lambda i, cu: (0, 0)),
            pl.BlockSpec((1, LAT), lambda i, cu: (0, 0)),
        ],
        out_specs=pl.BlockSpec((B, LAT), lambda i, cu: (0, 0)),
        scratch_shapes=[
            pltpu.VMEM((B, HID), jnp.float32),
            pltpu.VMEM((NELEM, HID), jnp.bfloat16),
            pltpu.VMEM((2, B), jnp.int32),
        ],
    )
    return pl.pallas_call(
        _fused_kernel,
        grid_spec=grid_spec,
        out_shape=jax.ShapeDtypeStruct((B, LAT), jnp.float32),
        compiler_params=pltpu.CompilerParams(
            dimension_semantics=("arbitrary",)),
    )(cu_seqlens, flat, W1, b1r, W2, b2r)
